# Initial kernel scaffold; baseline (speedup 1.0000x reference)
import jax
import jax.numpy as jnp
from jax.experimental import pallas as pl


def kernel(x, table):
    raise NotImplementedError("write your pallas kernel here")


# speedup vs baseline: 1.7965x; 1.7965x over previous
import functools

import jax
import jax.numpy as jnp
from jax import lax
from jax.experimental import pallas as pl
from jax.experimental.pallas import tpu as pltpu
from jax.experimental.pallas import tpu_sc as plsc


def _gather_call(N, D, n_chunks, chunk, mesh, num_cores):
    b_per_w = n_chunks * chunk

    @functools.partial(
        pl.kernel,
        mesh=mesh,
        compiler_params=pltpu.CompilerParams(use_tc_tiling_on_sc=False),
        out_type=jax.ShapeDtypeStruct((N, D), jnp.float32),
        scratch_types=[
            pltpu.VMEM((chunk,), jnp.int32),
            pltpu.VMEM((chunk, D), jnp.float32),
            pltpu.SemaphoreType.DMA,
        ],
    )
    def k(idx_hbm, tbl_hbm, out_hbm, idx_v, rows_v, sem):
        wid = lax.axis_index("s") * num_cores + lax.axis_index("c")
        base = wid * b_per_w

        def body(i, carry):
            off = base + i * chunk
            pltpu.sync_copy(idx_hbm.at[pl.ds(off, chunk)], idx_v)
            pltpu.async_copy(tbl_hbm.at[idx_v], rows_v, sem).wait()
            pltpu.sync_copy(rows_v, out_hbm.at[pl.ds(off, chunk)])
            return carry

        lax.fori_loop(0, n_chunks, body, 0)

    return k


def kernel(x, table):
    B, H = x.shape
    V, D = table.shape
    N = B * H
    idx = x.reshape(N).astype(jnp.int32)

    info = plsc.get_sparse_core_info()
    num_workers = info.num_cores * info.num_subcores
    b_per_w = N // num_workers
    chunk = 512
    n_chunks = b_per_w // chunk

    mesh = plsc.VectorSubcoreMesh(core_axis_name="c", subcore_axis_name="s")
    out = _gather_call(N, D, n_chunks, chunk, mesh, info.num_cores)(idx, table)
    return out.reshape(B, H, D)


# speedup vs baseline: 1.8754x; 1.0439x over previous
import functools

import jax
import jax.numpy as jnp
from jax import lax
from jax.experimental import pallas as pl
from jax.experimental.pallas import tpu as pltpu
from jax.experimental.pallas import tpu_sc as plsc


def _gather_call(N, D, b_per_w, C, nbuf, mesh, num_cores):
    n_chunks = b_per_w // C
    assert n_chunks % nbuf == 0 and n_chunks >= 2 * nbuf

    @functools.partial(
        pl.kernel,
        mesh=mesh,
        compiler_params=pltpu.CompilerParams(use_tc_tiling_on_sc=False),
        out_type=jax.ShapeDtypeStruct((N, D), jnp.float32),
        scratch_types=[
            pltpu.VMEM((b_per_w,), jnp.int32),
            pltpu.VMEM((nbuf, C, D), jnp.float32),
            pltpu.SemaphoreType.DMA((nbuf,)),
            pltpu.SemaphoreType.DMA((nbuf,)),
        ],
    )
    def k(idx_hbm, tbl_hbm, out_hbm, idx_v, rows_v, gsem, osem):
        wid = lax.axis_index("s") * num_cores + lax.axis_index("c")
        base = wid * b_per_w
        pltpu.sync_copy(idx_hbm.at[pl.ds(base, b_per_w)], idx_v)

        def gdesc(c, b):
            return pltpu.make_async_copy(
                tbl_hbm.at[idx_v.at[pl.ds(c * C, C)]], rows_v.at[b], gsem.at[b]
            )

        def odesc(c, b):
            return pltpu.make_async_copy(
                rows_v.at[b], out_hbm.at[pl.ds(base + c * C, C)], osem.at[b]
            )

        for b in range(nbuf):
            gdesc(b, b).start()

        def body(i, carry):
            i0 = i * nbuf
            for b in range(nbuf):
                gdesc(i0 + b, b).wait()
                odesc(i0 + b, b).start()
            for b in range(nbuf):
                odesc(i0 + b, b).wait()
                gdesc(i0 + b + nbuf, b).start()
            return carry

        lax.fori_loop(0, (n_chunks - nbuf) // nbuf, body, 0)

        c0 = n_chunks - nbuf
        for b in range(nbuf):
            gdesc(c0 + b, b).wait()
            odesc(c0 + b, b).start()
        for b in range(nbuf):
            odesc(c0 + b, b).wait()

    return k


def kernel(x, table):
    B, H = x.shape
    V, D = table.shape
    N = B * H
    idx = x.reshape(N).astype(jnp.int32)

    info = plsc.get_sparse_core_info()
    num_workers = info.num_cores * info.num_subcores
    b_per_w = N // num_workers

    mesh = plsc.VectorSubcoreMesh(core_axis_name="c", subcore_axis_name="s")
    out = _gather_call(N, D, b_per_w, 512, 2, mesh, info.num_cores)(idx, table)
    return out.reshape(B, H, D)
